# baseline (speedup 1.0000x reference)
import functools

import jax
import jax.numpy as jnp
from jax import lax
from jax.experimental import pallas as pl
from jax.experimental.pallas import tpu as pltpu
from jax.experimental.pallas import tpu_sc as plsc

_B = 16384
_D = 16
_V = 1000000
_NC = 2
_NS = 16
_NW = _NC * _NS
_BPW = _B // _NW
_REG = 0.001

_TCOLS = 4096
_TGRID = -(-_V // _TCOLS)
_PV = _TGRID * _TCOLS
_CS = _TCOLS // 8


def _relayout_body(p_in, q_in, r_in, p_out, q_out, r_out):
    for in_r, out_r in ((p_in, p_out), (q_in, q_out), (r_in, r_out)):
        xt = in_r[...].T
        for s in range(8):
            out_r[:, s * _D:(s + 1) * _D] = xt[s * _CS:(s + 1) * _CS, :]


_relayout3 = pl.pallas_call(
    _relayout_body,
    grid=(_TGRID,),
    in_specs=[pl.BlockSpec((_D, _TCOLS), lambda i: (0, i))] * 3,
    out_specs=[pl.BlockSpec((_TCOLS // 8, 8 * _D), lambda i: (i, 0))] * 3,
    out_shape=[jax.ShapeDtypeStruct((_PV // 8, 8 * _D), jnp.float32)] * 3,
)


def _gather3_body(ps_h, qs_h, rs_h, pe_t, qe_t, re_t,
                  pe_o, qe_o, re_o,
                  ip, iq, ir, rp, rq, rr, sp, sq, sr):
    wid = lax.axis_index("s") * _NC + lax.axis_index("c")
    base = wid * _BPW
    pltpu.sync_copy(ps_h.at[pl.ds(base, _BPW)], ip)
    pltpu.sync_copy(qs_h.at[pl.ds(base, _BPW)], iq)
    pltpu.sync_copy(rs_h.at[pl.ds(base, _BPW)], ir)

    def _remap(c, _):
        for ref in (ip, iq, ir):
            v = ref[pl.ds(c * 16, 16)]
            ref[pl.ds(c * 16, 16)] = (
                ((v >> 12) << 12) + ((v & (_CS - 1)) << 3) + ((v >> 9) & 7))
        return 0

    lax.fori_loop(0, _BPW // 16, _remap, 0)
    cp = pltpu.async_copy(pe_t.at[ip], rp, sp)
    cq = pltpu.async_copy(qe_t.at[iq], rq, sq)
    cr = pltpu.async_copy(re_t.at[ir], rr, sr)
    cp.wait()
    pltpu.sync_copy(rp, pe_o.at[pl.ds(base, _BPW)])
    cq.wait()
    pltpu.sync_copy(rq, qe_o.at[pl.ds(base, _BPW)])
    cr.wait()
    pltpu.sync_copy(rr, re_o.at[pl.ds(base, _BPW)])


@functools.cache
def _gather3():
    return pl.kernel(
        _gather3_body,
        out_type=[jax.ShapeDtypeStruct((_B, _D), jnp.float32)] * 3,
        mesh=plsc.VectorSubcoreMesh(core_axis_name="c", subcore_axis_name="s"),
        scratch_types=(
            [pltpu.VMEM((_BPW,), jnp.int32)] * 3
            + [pltpu.VMEM((_BPW, _D), jnp.float32)] * 3
            + [pltpu.SemaphoreType.DMA] * 3
        ),
        compiler_params=pltpu.CompilerParams(use_tc_tiling_on_sc=False),
    )


_BLK = 2048
_NBLK = _B // _BLK


def _dense_body(pe_r, qe_r, re_r, ww_r, wb_r, fcw_r, inf_r, regs_r, acc_r):
    i = pl.program_id(0)
    pe = pe_r[...]
    qe = qe_r[...]
    re = re_r[...]
    ww = ww_r[...]
    wb = wb_r[...]
    fcw = fcw_r[...]
    wc = ww / jnp.maximum(
        jnp.sqrt(jnp.sum(ww * ww, axis=1, keepdims=True)), 1.0)
    fcc = fcw / jnp.maximum(
        jnp.sqrt(jnp.sum(fcw * fcw, axis=1, keepdims=True)), 1.0)
    dot = functools.partial(
        lax.dot_general,
        dimension_numbers=(((1,), (1,)), ((), ())),
        precision=lax.Precision.HIGHEST,
        preferred_element_type=jnp.float32,
    )
    mlp = dot(pe, wc[:, 0:16]) + dot(qe, wc[:, 16:32]) + dot(re, wc[:, 32:48])
    h = jnp.maximum(pe * qe * re + mlp + wb, 0.0)
    inf_r[...] = jnp.sum(h * fcc, axis=1, keepdims=True)
    row = jnp.concatenate(
        [jnp.sum(pe * pe, axis=(0, 1), keepdims=True),
         jnp.sum(qe * qe, axis=(0, 1), keepdims=True),
         jnp.sum(re * re, axis=(0, 1), keepdims=True)], axis=1)

    @pl.when(i == 0)
    def _():
        acc_r[...] = row

    @pl.when(i > 0)
    def _():
        acc_r[...] += row

    @pl.when(i == _NBLK - 1)
    def _():
        acc = acc_r[...]
        regs_r[...] = _REG * (jnp.sqrt(acc[:, 0:1])
                              + jnp.sqrt(acc[:, 1:2])
                              + jnp.sqrt(acc[:, 2:3]))


_dense = pl.pallas_call(
    _dense_body,
    grid=(_NBLK,),
    in_specs=[
        pl.BlockSpec((_BLK, _D), lambda i: (i, 0)),
        pl.BlockSpec((_BLK, _D), lambda i: (i, 0)),
        pl.BlockSpec((_BLK, _D), lambda i: (i, 0)),
        pl.BlockSpec((_D, 3 * _D), lambda i: (0, 0)),
        pl.BlockSpec((1, _D), lambda i: (0, 0)),
        pl.BlockSpec((1, _D), lambda i: (0, 0)),
    ],
    out_specs=[
        pl.BlockSpec((_BLK, 1), lambda i: (i, 0)),
        pl.BlockSpec((1, 1), lambda i: (0, 0)),
    ],
    out_shape=[
        jax.ShapeDtypeStruct((_B, 1), jnp.float32),
        jax.ShapeDtypeStruct((1, 1), jnp.float32),
    ],
    scratch_shapes=[pltpu.VMEM((1, 3), jnp.float32)],
)


def kernel(ps, qs, rs, Pe, Qe, Re, W_w, W_b, FC_w):
    ps = ps.astype(jnp.int32)
    qs = qs.astype(jnp.int32)
    rs = rs.astype(jnp.int32)
    pe_lin, qe_lin, re_lin = (
        t.reshape(_PV, _D) for t in _relayout3(Pe.T, Qe.T, Re.T))
    pe, qe, re = _gather3()(ps, qs, rs, pe_lin, qe_lin, re_lin)
    inf, regs = _dense(pe, qe, re, W_w, W_b.reshape(1, _D), FC_w)
    return inf, regs.reshape(())


# speedup vs baseline: 1.0153x; 1.0153x over previous
import functools

import jax
import jax.numpy as jnp
from jax import lax
from jax.experimental import pallas as pl
from jax.experimental.pallas import tpu as pltpu
from jax.experimental.pallas import tpu_sc as plsc

_B = 16384
_D = 16
_V = 1000000
_NC = 2
_NS = 16
_NW = _NC * _NS
_BPW = _B // _NW
_REG = 0.001

_TCOLS = 8192
_TGRID = -(-_V // _TCOLS)
_PV = _TGRID * _TCOLS
_CS = _TCOLS // 8
_SHB = _TCOLS.bit_length() - 1
_SHC = _CS.bit_length() - 1


def _relayout_body(p_in, q_in, r_in, p_out, q_out, r_out):
    for in_r, out_r in ((p_in, p_out), (q_in, q_out), (r_in, r_out)):
        xt = in_r[...].T
        for s in range(8):
            out_r[:, s * _D:(s + 1) * _D] = xt[s * _CS:(s + 1) * _CS, :]


_relayout3 = pl.pallas_call(
    _relayout_body,
    grid=(_TGRID,),
    in_specs=[pl.BlockSpec((_D, _TCOLS), lambda i: (0, i))] * 3,
    out_specs=[pl.BlockSpec((_TCOLS // 8, 8 * _D), lambda i: (i, 0))] * 3,
    out_shape=[jax.ShapeDtypeStruct((_PV // 8, 8 * _D), jnp.float32)] * 3,
)


def _gather3_body(ps_h, qs_h, rs_h, pe_t, qe_t, re_t,
                  pe_o, qe_o, re_o,
                  ip, iq, ir, rp, rq, rr, sp, sq, sr):
    wid = lax.axis_index("s") * _NC + lax.axis_index("c")
    base = wid * _BPW
    pltpu.sync_copy(ps_h.at[pl.ds(base, _BPW)], ip)
    pltpu.sync_copy(qs_h.at[pl.ds(base, _BPW)], iq)
    pltpu.sync_copy(rs_h.at[pl.ds(base, _BPW)], ir)

    def _remap(c, _):
        for ref in (ip, iq, ir):
            v = ref[pl.ds(c * 16, 16)]
            ref[pl.ds(c * 16, 16)] = (
                ((v >> _SHB) << _SHB) + ((v & (_CS - 1)) << 3)
                + ((v >> _SHC) & 7))
        return 0

    lax.fori_loop(0, _BPW // 16, _remap, 0)
    cp = pltpu.async_copy(pe_t.at[ip], rp, sp)
    cq = pltpu.async_copy(qe_t.at[iq], rq, sq)
    cr = pltpu.async_copy(re_t.at[ir], rr, sr)
    cp.wait()
    pltpu.sync_copy(rp, pe_o.at[pl.ds(base, _BPW)])
    cq.wait()
    pltpu.sync_copy(rq, qe_o.at[pl.ds(base, _BPW)])
    cr.wait()
    pltpu.sync_copy(rr, re_o.at[pl.ds(base, _BPW)])


@functools.cache
def _gather3():
    return pl.kernel(
        _gather3_body,
        out_type=[jax.ShapeDtypeStruct((_B, _D), jnp.float32)] * 3,
        mesh=plsc.VectorSubcoreMesh(core_axis_name="c", subcore_axis_name="s"),
        scratch_types=(
            [pltpu.VMEM((_BPW,), jnp.int32)] * 3
            + [pltpu.VMEM((_BPW, _D), jnp.float32)] * 3
            + [pltpu.SemaphoreType.DMA] * 3
        ),
        compiler_params=pltpu.CompilerParams(use_tc_tiling_on_sc=False),
    )


_BLK = 2048
_NBLK = _B // _BLK


def _dense_body(pe_r, qe_r, re_r, ww_r, wb_r, fcw_r, inf_r, regs_r, acc_r):
    i = pl.program_id(0)
    pe = pe_r[...]
    qe = qe_r[...]
    re = re_r[...]
    ww = ww_r[...]
    wb = wb_r[...]
    fcw = fcw_r[...]
    wc = ww / jnp.maximum(
        jnp.sqrt(jnp.sum(ww * ww, axis=1, keepdims=True)), 1.0)
    fcc = fcw / jnp.maximum(
        jnp.sqrt(jnp.sum(fcw * fcw, axis=1, keepdims=True)), 1.0)
    dot = functools.partial(
        lax.dot_general,
        dimension_numbers=(((1,), (1,)), ((), ())),
        precision=lax.Precision.HIGHEST,
        preferred_element_type=jnp.float32,
    )
    mlp = dot(pe, wc[:, 0:16]) + dot(qe, wc[:, 16:32]) + dot(re, wc[:, 32:48])
    h = jnp.maximum(pe * qe * re + mlp + wb, 0.0)
    inf_r[...] = jnp.sum(h * fcc, axis=1, keepdims=True)
    row = jnp.concatenate(
        [jnp.sum(pe * pe, axis=(0, 1), keepdims=True),
         jnp.sum(qe * qe, axis=(0, 1), keepdims=True),
         jnp.sum(re * re, axis=(0, 1), keepdims=True)], axis=1)

    @pl.when(i == 0)
    def _():
        acc_r[...] = row

    @pl.when(i > 0)
    def _():
        acc_r[...] += row

    @pl.when(i == _NBLK - 1)
    def _():
        acc = acc_r[...]
        regs_r[...] = _REG * (jnp.sqrt(acc[:, 0:1])
                              + jnp.sqrt(acc[:, 1:2])
                              + jnp.sqrt(acc[:, 2:3]))


_dense = pl.pallas_call(
    _dense_body,
    grid=(_NBLK,),
    in_specs=[
        pl.BlockSpec((_BLK, _D), lambda i: (i, 0)),
        pl.BlockSpec((_BLK, _D), lambda i: (i, 0)),
        pl.BlockSpec((_BLK, _D), lambda i: (i, 0)),
        pl.BlockSpec((_D, 3 * _D), lambda i: (0, 0)),
        pl.BlockSpec((1, _D), lambda i: (0, 0)),
        pl.BlockSpec((1, _D), lambda i: (0, 0)),
    ],
    out_specs=[
        pl.BlockSpec((_BLK, 1), lambda i: (i, 0)),
        pl.BlockSpec((1, 1), lambda i: (0, 0)),
    ],
    out_shape=[
        jax.ShapeDtypeStruct((_B, 1), jnp.float32),
        jax.ShapeDtypeStruct((1, 1), jnp.float32),
    ],
    scratch_shapes=[pltpu.VMEM((1, 3), jnp.float32)],
)


def kernel(ps, qs, rs, Pe, Qe, Re, W_w, W_b, FC_w):
    ps = ps.astype(jnp.int32)
    qs = qs.astype(jnp.int32)
    rs = rs.astype(jnp.int32)
    pe_lin, qe_lin, re_lin = (
        t.reshape(_PV, _D) for t in _relayout3(Pe.T, Qe.T, Re.T))
    pe, qe, re = _gather3()(ps, qs, rs, pe_lin, qe_lin, re_lin)
    inf, regs = _dense(pe, qe, re, W_w, W_b.reshape(1, _D), FC_w)
    return inf, regs.reshape(())


# speedup vs baseline: 1.0176x; 1.0023x over previous
import functools

import jax
import jax.numpy as jnp
from jax import lax
from jax.experimental import pallas as pl
from jax.experimental.pallas import tpu as pltpu
from jax.experimental.pallas import tpu_sc as plsc

_B = 16384
_D = 16
_V = 1000000
_NC = 2
_NS = 16
_NW = _NC * _NS
_BPW = _B // _NW
_REG = 0.001

_TCOLS = 16384
_TGRID = -(-_V // _TCOLS)
_PV = _TGRID * _TCOLS
_CS = _TCOLS // 8
_SHB = _TCOLS.bit_length() - 1
_SHC = _CS.bit_length() - 1


def _relayout_body(p_in, q_in, r_in, p_out, q_out, r_out):
    for in_r, out_r in ((p_in, p_out), (q_in, q_out), (r_in, r_out)):
        xt = in_r[...].T
        for s in range(8):
            out_r[:, s * _D:(s + 1) * _D] = xt[s * _CS:(s + 1) * _CS, :]


_relayout3 = pl.pallas_call(
    _relayout_body,
    grid=(_TGRID,),
    in_specs=[pl.BlockSpec((_D, _TCOLS), lambda i: (0, i))] * 3,
    out_specs=[pl.BlockSpec((_TCOLS // 8, 8 * _D), lambda i: (i, 0))] * 3,
    out_shape=[jax.ShapeDtypeStruct((_PV // 8, 8 * _D), jnp.float32)] * 3,
)


def _gather3_body(ps_h, qs_h, rs_h, pe_t, qe_t, re_t,
                  pe_o, qe_o, re_o,
                  ip, iq, ir, rp, rq, rr, sp, sq, sr):
    wid = lax.axis_index("s") * _NC + lax.axis_index("c")
    base = wid * _BPW
    pltpu.sync_copy(ps_h.at[pl.ds(base, _BPW)], ip)
    pltpu.sync_copy(qs_h.at[pl.ds(base, _BPW)], iq)
    pltpu.sync_copy(rs_h.at[pl.ds(base, _BPW)], ir)

    def _remap(c, _):
        for ref in (ip, iq, ir):
            v = ref[pl.ds(c * 16, 16)]
            ref[pl.ds(c * 16, 16)] = (
                ((v >> _SHB) << _SHB) + ((v & (_CS - 1)) << 3)
                + ((v >> _SHC) & 7))
        return 0

    lax.fori_loop(0, _BPW // 16, _remap, 0)
    cp = pltpu.async_copy(pe_t.at[ip], rp, sp)
    cq = pltpu.async_copy(qe_t.at[iq], rq, sq)
    cr = pltpu.async_copy(re_t.at[ir], rr, sr)
    cp.wait()
    pltpu.sync_copy(rp, pe_o.at[pl.ds(base, _BPW)])
    cq.wait()
    pltpu.sync_copy(rq, qe_o.at[pl.ds(base, _BPW)])
    cr.wait()
    pltpu.sync_copy(rr, re_o.at[pl.ds(base, _BPW)])


@functools.cache
def _gather3():
    return pl.kernel(
        _gather3_body,
        out_type=[jax.ShapeDtypeStruct((_B, _D), jnp.float32)] * 3,
        mesh=plsc.VectorSubcoreMesh(core_axis_name="c", subcore_axis_name="s"),
        scratch_types=(
            [pltpu.VMEM((_BPW,), jnp.int32)] * 3
            + [pltpu.VMEM((_BPW, _D), jnp.float32)] * 3
            + [pltpu.SemaphoreType.DMA] * 3
        ),
        compiler_params=pltpu.CompilerParams(use_tc_tiling_on_sc=False),
    )


_BLK = 2048
_NBLK = _B // _BLK


def _dense_body(pe_r, qe_r, re_r, ww_r, wb_r, fcw_r, inf_r, regs_r, acc_r):
    i = pl.program_id(0)
    pe = pe_r[...]
    qe = qe_r[...]
    re = re_r[...]
    ww = ww_r[...]
    wb = wb_r[...]
    fcw = fcw_r[...]
    wc = ww / jnp.maximum(
        jnp.sqrt(jnp.sum(ww * ww, axis=1, keepdims=True)), 1.0)
    fcc = fcw / jnp.maximum(
        jnp.sqrt(jnp.sum(fcw * fcw, axis=1, keepdims=True)), 1.0)
    dot = functools.partial(
        lax.dot_general,
        dimension_numbers=(((1,), (1,)), ((), ())),
        precision=lax.Precision.HIGHEST,
        preferred_element_type=jnp.float32,
    )
    mlp = dot(pe, wc[:, 0:16]) + dot(qe, wc[:, 16:32]) + dot(re, wc[:, 32:48])
    h = jnp.maximum(pe * qe * re + mlp + wb, 0.0)
    inf_r[...] = jnp.sum(h * fcc, axis=1, keepdims=True)
    row = jnp.concatenate(
        [jnp.sum(pe * pe, axis=(0, 1), keepdims=True),
         jnp.sum(qe * qe, axis=(0, 1), keepdims=True),
         jnp.sum(re * re, axis=(0, 1), keepdims=True)], axis=1)

    @pl.when(i == 0)
    def _():
        acc_r[...] = row

    @pl.when(i > 0)
    def _():
        acc_r[...] += row

    @pl.when(i == _NBLK - 1)
    def _():
        acc = acc_r[...]
        regs_r[...] = _REG * (jnp.sqrt(acc[:, 0:1])
                              + jnp.sqrt(acc[:, 1:2])
                              + jnp.sqrt(acc[:, 2:3]))


_dense = pl.pallas_call(
    _dense_body,
    grid=(_NBLK,),
    in_specs=[
        pl.BlockSpec((_BLK, _D), lambda i: (i, 0)),
        pl.BlockSpec((_BLK, _D), lambda i: (i, 0)),
        pl.BlockSpec((_BLK, _D), lambda i: (i, 0)),
        pl.BlockSpec((_D, 3 * _D), lambda i: (0, 0)),
        pl.BlockSpec((1, _D), lambda i: (0, 0)),
        pl.BlockSpec((1, _D), lambda i: (0, 0)),
    ],
    out_specs=[
        pl.BlockSpec((_BLK, 1), lambda i: (i, 0)),
        pl.BlockSpec((1, 1), lambda i: (0, 0)),
    ],
    out_shape=[
        jax.ShapeDtypeStruct((_B, 1), jnp.float32),
        jax.ShapeDtypeStruct((1, 1), jnp.float32),
    ],
    scratch_shapes=[pltpu.VMEM((1, 3), jnp.float32)],
)


def kernel(ps, qs, rs, Pe, Qe, Re, W_w, W_b, FC_w):
    ps = ps.astype(jnp.int32)
    qs = qs.astype(jnp.int32)
    rs = rs.astype(jnp.int32)
    pe_lin, qe_lin, re_lin = (
        t.reshape(_PV, _D) for t in _relayout3(Pe.T, Qe.T, Re.T))
    pe, qe, re = _gather3()(ps, qs, rs, pe_lin, qe_lin, re_lin)
    inf, regs = _dense(pe, qe, re, W_w, W_b.reshape(1, _D), FC_w)
    return inf, regs.reshape(())
